# baseline (speedup 1.0000x reference)
import functools

import jax
import jax.numpy as jnp
from jax import lax
from jax.experimental import pallas as pl
from jax.experimental.pallas import tpu as pltpu
from jax.experimental.pallas import tpu_sc as plsc

_NC = 2
_NS = 16
_NW = _NC * _NS
_L = 16


def _wc_body(wv_ref, bv_ref, wo_ref, wc_ref, *, C):
    wc_ref[pl.ds(0, C), :] = jnp.dot(wv_ref[...].astype(jnp.bfloat16),
                                     wo_ref[...].astype(jnp.bfloat16),
                                     preferred_element_type=jnp.float32)
    brow = jnp.dot(bv_ref[...], wo_ref[...],
                   preferred_element_type=jnp.float32)
    wc_ref[pl.ds(C, 8), :] = jnp.broadcast_to(brow, (8, C))


def _prep_body(q_ref, w_ref, b_ref, tref_ref, idx_ref,
               coef_ref, rowid_ref, *, K, T, Lv, bconst):
    proj = jnp.dot(q_ref[...], w_ref[...],
                   preferred_element_type=jnp.float32) + b_ref[...]
    off = proj[:, :K]
    logit = proj[:, K:]
    m = jnp.max(logit, axis=1, keepdims=True)
    e = jnp.exp(logit - m)
    w = e / jnp.sum(e, axis=1, keepdims=True)
    ts = jnp.clip(tref_ref[...] + off, 0.0, float(T - 1))
    t0f = jnp.floor(ts)
    t0 = t0f.astype(jnp.int32)
    t1 = jnp.minimum(t0 + 1, T - 1)
    a = ts - t0f
    w0 = w * (1.0 - a)
    w1 = w * a
    NB = q_ref.shape[0]
    tt = lax.broadcasted_iota(jnp.int32, (NB, T), 1)
    coef = jnp.zeros((NB, T), jnp.float32)
    for k in range(K):
        coef = (coef
                + jnp.where(t0[:, k:k + 1] == tt, w0[:, k:k + 1], 0.0)
                + jnp.where(t1[:, k:k + 1] == tt, w1[:, k:k + 1], 0.0))
    coef_ref[...] = coef
    rowid_ref[...] = (bconst * T + tt) * Lv + idx_ref[...]


def _epi_body(a_ref, w_ref, bout_ref, *rest, Kdim):
    o_ref = rest[-1]
    acc = jnp.dot(a_ref[...].astype(jnp.bfloat16),
                  w_ref[pl.ds(0, Kdim), :].astype(jnp.bfloat16),
                  preferred_element_type=jnp.float32)
    o_ref[0] = acc + w_ref[Kdim, :][None, :] + bout_ref[...]


def _make_sc_gather(C, N, T):
    QT = N // _NW
    QB = 4
    NR = QT // QB
    QBT = QB * T
    CH = C // _L
    assert QT % QB == 0 and NR % 2 == 0 and C % _L == 0

    mesh = plsc.VectorSubcoreMesh(core_axis_name="c", subcore_axis_name="s")

    @functools.partial(
        pl.kernel,
        out_type=jax.ShapeDtypeStruct((N, C), jnp.float32),
        mesh=mesh,
        scratch_types=[
            pltpu.VMEM((QT * T,), jnp.int32),
            pltpu.VMEM((QT * T,), jnp.float32),
            pltpu.VMEM((2, QBT, C), jnp.float32),
            pltpu.VMEM((2, QB, C), jnp.float32),
            pltpu.SemaphoreType.DMA,
            pltpu.SemaphoreType.DMA,
            pltpu.SemaphoreType.DMA,
            pltpu.SemaphoreType.DMA,
        ],
    )
    def sc_gather(fp_hbm, ids_hbm, coef_hbm, out_hbm,
                  ids_v, coef_v, gbuf, obuf, gsem0, gsem1, osem0, osem1):
        cid = lax.axis_index("c")
        sid = lax.axis_index("s")
        wid = sid * _NC + cid
        qbase = wid * QT
        pltpu.sync_copy(ids_hbm.at[pl.ds(qbase * T, QT * T)], ids_v)
        pltpu.sync_copy(coef_hbm.at[pl.ds(qbase * T, QT * T)], coef_v)
        gsems = (gsem0, gsem1)
        osems = (osem0, osem1)

        def g_desc(rr, ph):
            return pltpu.make_async_copy(
                fp_hbm.at[ids_v.at[pl.ds(rr * QBT, QBT)]],
                gbuf.at[ph], gsems[ph])

        def o_desc(rr, ph):
            return pltpu.make_async_copy(
                obuf.at[ph], out_hbm.at[pl.ds(qbase + rr * QB, QB)],
                osems[ph])

        g_desc(0, 0).start()

        @pl.loop(0, NR, step=2)
        def _round(r):
            for ph in range(2):
                rr = r + ph
                g_desc(rr, ph).wait()

                @pl.when(rr + 1 < NR)
                def _():
                    g_desc(rr + 1, 1 - ph).start()

                @pl.when(rr >= 2)
                def _():
                    o_desc(rr - 2, ph).wait()

                for jp in range(QB // 2):
                    crow = coef_v[pl.ds((rr * QB + 2 * jp) * T, 2 * T)]
                    for h in range(2):
                        j = 2 * jp + h
                        c = [crow[h * T + t] for t in range(T)]

                        @plsc.parallel_loop(0, CH, unroll=8)
                        def _chunk(i):
                            s = pl.ds(i * _L, _L)
                            a0 = gbuf[ph, j * T + 0, s] * c[0] + gbuf[ph, j * T + 1, s] * c[1]
                            a1 = gbuf[ph, j * T + 2, s] * c[2] + gbuf[ph, j * T + 3, s] * c[3]
                            a2 = gbuf[ph, j * T + 4, s] * c[4] + gbuf[ph, j * T + 5, s] * c[5]
                            a3 = gbuf[ph, j * T + 6, s] * c[6] + gbuf[ph, j * T + 7, s] * c[7]
                            obuf[ph, j, s] = (a0 + a1) + (a2 + a3)

                o_desc(rr, ph).start()

        o_desc(NR - 2, 0).wait()
        o_desc(NR - 1, 1).wait()

    return sc_gather


def kernel(Fp, queries, idx_q, t_ref_q, W_off, b_off, W_attn, b_attn,
           W_v, b_v, W_out, b_out):
    B, T, Lv, C = Fp.shape
    Lq = queries.shape[1]
    K = W_off.shape[1]
    N = B * Lq

    q2 = queries.reshape(N, C)
    wcat = jnp.concatenate([W_off, W_attn], axis=1)
    bcat = jnp.concatenate([b_off, b_attn])[None, :]
    tref2 = t_ref_q.reshape(N, 1)
    idx2 = idx_q.reshape(N, 1)

    halves = []
    for bb in range(B):
        halves.append(pl.pallas_call(
            functools.partial(_prep_body, K=K, T=T, Lv=Lv, bconst=bb),
            grid=(1,),
            in_specs=[pl.BlockSpec((Lq, C), lambda i, bb=bb: (bb, 0)),
                      pl.BlockSpec((C, 2 * K), lambda i: (0, 0)),
                      pl.BlockSpec((1, 2 * K), lambda i: (0, 0)),
                      pl.BlockSpec((Lq, 1), lambda i, bb=bb: (bb, 0)),
                      pl.BlockSpec((Lq, 1), lambda i, bb=bb: (bb, 0))],
            out_specs=(pl.BlockSpec((Lq, T), lambda i: (0, 0)),
                       pl.BlockSpec((Lq, T), lambda i: (0, 0))),
            out_shape=(jax.ShapeDtypeStruct((Lq, T), jnp.float32),
                       jax.ShapeDtypeStruct((Lq, T), jnp.int32)),
        )(q2, wcat, bcat, tref2, idx2))

    wc = pl.pallas_call(
        functools.partial(_wc_body, C=C),
        out_shape=jax.ShapeDtypeStruct((C + 8, C), jnp.float32),
    )(W_v, b_v[None, :], W_out)

    fp2 = Fp.reshape(B * T * Lv, C)
    NH = Lq
    sc = _make_sc_gather(C, NH, T)
    agg_a = sc(fp2, halves[0][1].reshape(NH * T), halves[0][0].reshape(NH * T))
    agg_b = sc(fp2, halves[1][1].reshape(NH * T), halves[1][0].reshape(NH * T))

    MB = 1024
    NBQ = Lq // MB
    out_a = pl.pallas_call(
        functools.partial(_epi_body, Kdim=C),
        grid=(NH // MB,),
        in_specs=[pl.BlockSpec((MB, C), lambda i: (i, 0)),
                  pl.BlockSpec((C + 8, C), lambda i: (0, 0)),
                  pl.BlockSpec((1, C), lambda i: (0, 0))],
        out_specs=pl.BlockSpec((1, MB, C), lambda i: (i // NBQ, i % NBQ, 0)),
        out_shape=jax.ShapeDtypeStruct((B, Lq, C), jnp.float32),
    )(agg_a, wc, b_out[None, :])
    out = pl.pallas_call(
        functools.partial(_epi_body, Kdim=C),
        grid=(NH // MB,),
        in_specs=[pl.BlockSpec((MB, C), lambda i: (i, 0)),
                  pl.BlockSpec((C + 8, C), lambda i: (0, 0)),
                  pl.BlockSpec((1, C), lambda i: (0, 0)),
                  pl.BlockSpec(memory_space=pltpu.HBM)],
        out_specs=pl.BlockSpec((1, MB, C),
                               lambda i: (1 + i // NBQ, i % NBQ, 0)),
        out_shape=jax.ShapeDtypeStruct((B, Lq, C), jnp.float32),
        input_output_aliases={3: 0},
    )(agg_b, wc, b_out[None, :], out_a)
    return out


# speedup vs baseline: 1.0027x; 1.0027x over previous
import functools

import jax
import jax.numpy as jnp
from jax import lax
from jax.experimental import pallas as pl
from jax.experimental.pallas import tpu as pltpu
from jax.experimental.pallas import tpu_sc as plsc

_NC = 2
_NS = 16
_NW = _NC * _NS
_L = 16


def _wc_body(wv_ref, bv_ref, wo_ref, wc_ref, *, C):
    wc_ref[pl.ds(0, C), :] = jnp.dot(wv_ref[...].astype(jnp.bfloat16),
                                     wo_ref[...].astype(jnp.bfloat16),
                                     preferred_element_type=jnp.float32)
    brow = jnp.dot(bv_ref[...], wo_ref[...],
                   preferred_element_type=jnp.float32)
    wc_ref[pl.ds(C, 8), :] = jnp.broadcast_to(brow, (8, C))


def _prep_body(q_ref, w_ref, b_ref, tref_ref, idx_ref,
               coef_ref, rowid_ref, *, K, T, Lv, bconst):
    proj = jnp.dot(q_ref[...], w_ref[...],
                   preferred_element_type=jnp.float32) + b_ref[...]
    off = proj[:, :K]
    logit = proj[:, K:]
    m = jnp.max(logit, axis=1, keepdims=True)
    e = jnp.exp(logit - m)
    w = e / jnp.sum(e, axis=1, keepdims=True)
    ts = jnp.clip(tref_ref[...] + off, 0.0, float(T - 1))
    t0f = jnp.floor(ts)
    t0 = t0f.astype(jnp.int32)
    t1 = jnp.minimum(t0 + 1, T - 1)
    a = ts - t0f
    w0 = w * (1.0 - a)
    w1 = w * a
    NB = q_ref.shape[0]
    tt = lax.broadcasted_iota(jnp.int32, (NB, T), 1)
    coef = jnp.zeros((NB, T), jnp.float32)
    for k in range(K):
        coef = (coef
                + jnp.where(t0[:, k:k + 1] == tt, w0[:, k:k + 1], 0.0)
                + jnp.where(t1[:, k:k + 1] == tt, w1[:, k:k + 1], 0.0))
    coef_ref[...] = coef
    rowid_ref[...] = (bconst * T + tt) * Lv + idx_ref[...]


def _epi_body(a_ref, w_ref, bout_ref, *rest, Kdim):
    o_ref = rest[-1]
    acc = jnp.dot(a_ref[...].astype(jnp.bfloat16),
                  w_ref[pl.ds(0, Kdim), :].astype(jnp.bfloat16),
                  preferred_element_type=jnp.float32)
    o_ref[0] = acc + w_ref[Kdim, :][None, :] + bout_ref[...]


def _make_sc_gather(C, N, T):
    QT = N // _NW
    QB = 4
    NR = QT // QB
    QBT = QB * T
    CH = C // _L
    assert QT % QB == 0 and NR % 2 == 0 and C % _L == 0

    mesh = plsc.VectorSubcoreMesh(core_axis_name="c", subcore_axis_name="s")

    @functools.partial(
        pl.kernel,
        out_type=jax.ShapeDtypeStruct((N, C), jnp.float32),
        mesh=mesh,
        scratch_types=[
            pltpu.VMEM((QT * T,), jnp.int32),
            pltpu.VMEM((QT * T,), jnp.float32),
            pltpu.VMEM((2, QBT, C), jnp.float32),
            pltpu.VMEM((2, QB, C), jnp.float32),
            pltpu.SemaphoreType.DMA,
            pltpu.SemaphoreType.DMA,
            pltpu.SemaphoreType.DMA,
            pltpu.SemaphoreType.DMA,
        ],
    )
    def sc_gather(fp_hbm, ids_hbm, coef_hbm, out_hbm,
                  ids_v, coef_v, gbuf, obuf, gsem0, gsem1, osem0, osem1):
        cid = lax.axis_index("c")
        sid = lax.axis_index("s")
        wid = sid * _NC + cid
        qbase = wid * QT
        pltpu.sync_copy(ids_hbm.at[pl.ds(qbase * T, QT * T)], ids_v)
        pltpu.sync_copy(coef_hbm.at[pl.ds(qbase * T, QT * T)], coef_v)
        gsems = (gsem0, gsem1)
        osems = (osem0, osem1)

        def g_desc(rr, ph):
            return pltpu.make_async_copy(
                fp_hbm.at[ids_v.at[pl.ds(rr * QBT, QBT)]],
                gbuf.at[ph], gsems[ph])

        def o_desc(rr, ph):
            return pltpu.make_async_copy(
                obuf.at[ph], out_hbm.at[pl.ds(qbase + rr * QB, QB)],
                osems[ph])

        g_desc(0, 0).start()

        @pl.loop(0, NR, step=2)
        def _round(r):
            for ph in range(2):
                rr = r + ph
                g_desc(rr, ph).wait()

                @pl.when(rr + 1 < NR)
                def _():
                    g_desc(rr + 1, 1 - ph).start()

                @pl.when(rr >= 2)
                def _():
                    o_desc(rr - 2, ph).wait()

                for jp in range(QB // 2):
                    crow = coef_v[pl.ds((rr * QB + 2 * jp) * T, 2 * T)]
                    for h in range(2):
                        j = 2 * jp + h
                        c = [crow[h * T + t] for t in range(T)]

                        @plsc.parallel_loop(0, CH, unroll=8)
                        def _chunk(i):
                            s = pl.ds(i * _L, _L)
                            a0 = gbuf[ph, j * T + 0, s] * c[0] + gbuf[ph, j * T + 1, s] * c[1]
                            a1 = gbuf[ph, j * T + 2, s] * c[2] + gbuf[ph, j * T + 3, s] * c[3]
                            a2 = gbuf[ph, j * T + 4, s] * c[4] + gbuf[ph, j * T + 5, s] * c[5]
                            a3 = gbuf[ph, j * T + 6, s] * c[6] + gbuf[ph, j * T + 7, s] * c[7]
                            obuf[ph, j, s] = (a0 + a1) + (a2 + a3)

                o_desc(rr, ph).start()

        o_desc(NR - 2, 0).wait()
        o_desc(NR - 1, 1).wait()

    return sc_gather


def kernel(Fp, queries, idx_q, t_ref_q, W_off, b_off, W_attn, b_attn,
           W_v, b_v, W_out, b_out):
    B, T, Lv, C = Fp.shape
    Lq = queries.shape[1]
    K = W_off.shape[1]
    N = B * Lq

    q2 = queries.reshape(N, C)
    wcat = jnp.concatenate([W_off, W_attn], axis=1)
    bcat = jnp.concatenate([b_off, b_attn])[None, :]
    tref2 = t_ref_q.reshape(N, 1)
    idx2 = idx_q.reshape(N, 1)

    halves = []
    for bb in range(B):
        halves.append(pl.pallas_call(
            functools.partial(_prep_body, K=K, T=T, Lv=Lv, bconst=bb),
            grid=(1,),
            in_specs=[pl.BlockSpec((Lq, C), lambda i, bb=bb: (bb, 0)),
                      pl.BlockSpec((C, 2 * K), lambda i: (0, 0)),
                      pl.BlockSpec((1, 2 * K), lambda i: (0, 0)),
                      pl.BlockSpec((Lq, 1), lambda i, bb=bb: (bb, 0)),
                      pl.BlockSpec((Lq, 1), lambda i, bb=bb: (bb, 0))],
            out_specs=(pl.BlockSpec((Lq, T), lambda i: (0, 0)),
                       pl.BlockSpec((Lq, T), lambda i: (0, 0))),
            out_shape=(jax.ShapeDtypeStruct((Lq, T), jnp.float32),
                       jax.ShapeDtypeStruct((Lq, T), jnp.int32)),
        )(q2, wcat, bcat, tref2, idx2))

    wc = pl.pallas_call(
        functools.partial(_wc_body, C=C),
        out_shape=jax.ShapeDtypeStruct((C + 8, C), jnp.float32),
    )(W_v, b_v[None, :], W_out)

    fp2 = Fp.reshape(B * T * Lv, C)
    NH = Lq
    sc = _make_sc_gather(C, NH, T)
    agg_a = sc(fp2, halves[0][1].reshape(NH * T), halves[0][0].reshape(NH * T))
    agg_b = sc(fp2, halves[1][1].reshape(NH * T), halves[1][0].reshape(NH * T))

    MB = 512
    NBQ = Lq // MB
    out_a = pl.pallas_call(
        functools.partial(_epi_body, Kdim=C),
        grid=(NH // MB,),
        in_specs=[pl.BlockSpec((MB, C), lambda i: (i, 0)),
                  pl.BlockSpec((C + 8, C), lambda i: (0, 0)),
                  pl.BlockSpec((1, C), lambda i: (0, 0))],
        out_specs=pl.BlockSpec((1, MB, C), lambda i: (i // NBQ, i % NBQ, 0)),
        out_shape=jax.ShapeDtypeStruct((B, Lq, C), jnp.float32),
    )(agg_a, wc, b_out[None, :])
    out = pl.pallas_call(
        functools.partial(_epi_body, Kdim=C),
        grid=(NH // MB,),
        in_specs=[pl.BlockSpec((MB, C), lambda i: (i, 0)),
                  pl.BlockSpec((C + 8, C), lambda i: (0, 0)),
                  pl.BlockSpec((1, C), lambda i: (0, 0)),
                  pl.BlockSpec(memory_space=pltpu.HBM)],
        out_specs=pl.BlockSpec((1, MB, C),
                               lambda i: (1 + i // NBQ, i % NBQ, 0)),
        out_shape=jax.ShapeDtypeStruct((B, Lq, C), jnp.float32),
        input_output_aliases={3: 0},
    )(agg_b, wc, b_out[None, :], out_a)
    return out
